# baseline (speedup 1.0000x reference)
import functools

import jax
import jax.numpy as jnp
from jax.experimental import pallas as pl

_BETA = 0.25


def _vq_body(nb, nk, hw, z_ref, cb_ref, zq_ref, idx_ref, loss_ref):
    cb = cb_ref[...]
    cnorm = jnp.sum(cb * cb, axis=1, keepdims=True)
    kiota = jax.lax.broadcasted_iota(jnp.int32, (nk, hw), 0)

    acc = jnp.zeros((1, 1), jnp.float32)
    for b in range(nb):
        z = z_ref[b]
        s = jax.lax.dot_general(
            cb, z * -2.0, (((1,), (0,)), ((), ())),
            preferred_element_type=jnp.float32)
        znorm = jnp.sum(z * z, axis=0, keepdims=True)
        dist = (znorm + s) + cnorm

        m = jnp.min(dist, axis=0, keepdims=True)
        idx = jnp.min(jnp.where(dist == m, kiota, nk), axis=0, keepdims=True)
        idx_ref[:, pl.ds(b * hw, hw)] = idx

        onehot = (kiota == idx).astype(jnp.float32)
        zq = jax.lax.dot_general(
            cb, onehot, (((0,), (0,)), ((), ())),
            preferred_element_type=jnp.float32)

        d = zq - z
        zq_ref[b] = z + d
        acc = acc + jnp.sum(d * d, keepdims=True)

    mean = acc / (nb * z_ref.shape[1] * hw)
    loss_ref[...] = _BETA * mean + mean


def kernel(z, codebook):
    B, C, H, W = z.shape
    K = codebook.shape[0]
    HW = H * W
    z3 = z.reshape(B, C, HW)

    zq3, idx2, loss11 = pl.pallas_call(
        functools.partial(_vq_body, B, K, HW),
        grid=(1,),
        in_specs=[
            pl.BlockSpec((B, C, HW), lambda i: (0, 0, 0)),
            pl.BlockSpec((K, C), lambda i: (0, 0)),
        ],
        out_specs=[
            pl.BlockSpec((B, C, HW), lambda i: (0, 0, 0)),
            pl.BlockSpec((1, B * HW), lambda i: (0, 0)),
            pl.BlockSpec((1, 1), lambda i: (0, 0)),
        ],
        out_shape=[
            jax.ShapeDtypeStruct((B, C, HW), jnp.float32),
            jax.ShapeDtypeStruct((1, B * HW), jnp.int32),
            jax.ShapeDtypeStruct((1, 1), jnp.float32),
        ],
    )(z3, codebook)

    zq = zq3.reshape(B, C, H, W)
    idx = idx2.reshape(-1)
    loss = loss11.reshape(())
    return zq, idx, loss
